# baseline (speedup 1.0000x reference)
import functools

import jax
import jax.numpy as jnp
from jax import lax
from jax.experimental import pallas as pl
from jax.experimental.pallas import tpu as pltpu
from jax.experimental.pallas import tpu_sc as plsc

NUM_CORES = 2
NUM_SUBCORES = 16
NUM_WORKERS = NUM_CORES * NUM_SUBCORES

CHUNK = 128
LANES = 16


def _gather(idx, weight, total, D, n_chunks):
    per_worker = total // NUM_WORKERS
    n_pairs = n_chunks // 2
    mesh = plsc.VectorSubcoreMesh(core_axis_name="c", subcore_axis_name="s")

    @functools.partial(
        pl.kernel,
        mesh=mesh,
        out_type=jax.ShapeDtypeStruct((total, D), jnp.float32),
        scratch_types=[
            pltpu.VMEM((n_chunks, CHUNK), jnp.int32),
            pltpu.VMEM((CHUNK, D), jnp.float32),
            pltpu.VMEM((CHUNK, D), jnp.float32),
            pltpu.SemaphoreType.DMA,
            pltpu.SemaphoreType.DMA,
        ],
        compiler_params=pltpu.CompilerParams(use_tc_tiling_on_sc=False),
    )
    def gather_kernel(idx_hbm, table_hbm, out_hbm, idx_v, rows_a, rows_b, sem_a, sem_b):
        wid = lax.axis_index("s") * NUM_CORES + lax.axis_index("c")
        base = wid * per_worker
        pltpu.sync_copy(idx_hbm.at[wid], idx_v)

        pltpu.async_copy(table_hbm.at[idx_v.at[0]], rows_a, sem_a)

        def body(i, carry):
            g = 2 * i
            pltpu.async_copy(table_hbm.at[idx_v.at[g + 1]], rows_b, sem_b)
            pltpu.make_async_copy(table_hbm.at[idx_v.at[g]], rows_a, sem_a).wait()
            pltpu.sync_copy(rows_a, out_hbm.at[pl.ds(base + g * CHUNK, CHUNK)])

            @pl.when(i < n_pairs - 1)
            def _():
                pltpu.async_copy(table_hbm.at[idx_v.at[g + 2]], rows_a, sem_a)

            pltpu.make_async_copy(table_hbm.at[idx_v.at[g + 1]], rows_b, sem_b).wait()
            pltpu.sync_copy(rows_b, out_hbm.at[pl.ds(base + (g + 1) * CHUNK, CHUNK)])
            return carry

        lax.fori_loop(0, n_pairs, body, 0)

    return gather_kernel(idx, weight)


def _transpose(pairs, B, S, D):
    P = S // 2
    nblk = B // 128
    n_units = P * nblk
    units_per_w = n_units // NUM_WORKERS
    mesh = plsc.VectorSubcoreMesh(core_axis_name="c", subcore_axis_name="s")

    @functools.partial(
        pl.kernel,
        mesh=mesh,
        out_type=jax.ShapeDtypeStruct((S, D, B), jnp.float32),
        scratch_types=[
            pltpu.VMEM((128, 128), jnp.float32),
            pltpu.VMEM((128, 128), jnp.float32),
            pltpu.VMEM((D, 128), jnp.float32),
            pltpu.VMEM((D, 128), jnp.float32),
            pltpu.SemaphoreType.DMA,
            pltpu.SemaphoreType.DMA,
            pltpu.SemaphoreType.DMA,
            pltpu.SemaphoreType.DMA,
        ],
        compiler_params=pltpu.CompilerParams(
            use_tc_tiling_on_sc=True, needs_layout_passes=False
        ),
    )
    def ckernel(in_hbm, out_hbm, in_a, in_b, tile_a, tile_b, sia, sib, soa, sob):
        wid = lax.axis_index("s") * NUM_CORES + lax.axis_index("c")
        ubase = wid * units_per_w

        def src(u):
            return in_hbm.at[u // nblk, pl.ds((u % nblk) * 128, 128)]

        pltpu.async_copy(src(ubase), in_a, sia)

        def run_unit(u, stage, sin, other, osin, first):
            p = u // nblk
            blk = u % nblk
            pltpu.make_async_copy(src(u), stage, sin).wait()

            @pl.when(u + 1 < ubase + units_per_w)
            def _():
                pltpu.async_copy(src(u + 1), other, osin)

            for h, (tbuf, osem) in enumerate(((tile_a, soa), (tile_b, sob))):
                if not first:
                    pltpu.make_async_copy(
                        tbuf, out_hbm.at[0, pl.ds(0, D), pl.ds(0, 128)], osem
                    ).wait()

                def trow(c, carry2):
                    def tgrp(j, carry3):
                        rowv = lax.iota(jnp.int32, 16) + j * LANES
                        colv = jnp.full((16,), h * D + c, dtype=jnp.int32)
                        vals = plsc.load_gather(stage, [rowv, colv])
                        tbuf[c, pl.ds(j * LANES, LANES)] = vals
                        return carry3

                    return lax.fori_loop(0, 128 // LANES, tgrp, carry2)

                lax.fori_loop(0, D, trow, 0)
                pltpu.async_copy(
                    tbuf,
                    out_hbm.at[2 * p + h, pl.ds(0, D), pl.ds(blk * 128, 128)],
                    osem,
                )

        run_unit(ubase, in_a, sia, in_b, sib, True)

        def body(i, carry):
            u = ubase + 1 + 2 * i
            run_unit(u, in_b, sib, in_a, sia, False)
            run_unit(u + 1, in_a, sia, in_b, sib, False)
            return carry

        lax.fori_loop(0, (units_per_w - 1) // 2, body, 0)
        if units_per_w % 2 == 0:
            run_unit(ubase + units_per_w - 1, in_b, sib, in_a, sia, False)
        for tbuf, osem in ((tile_a, soa), (tile_b, sob)):
            pltpu.make_async_copy(
                tbuf, out_hbm.at[0, pl.ds(0, D), pl.ds(0, 128)], osem
            ).wait()

    return ckernel(pairs)


def kernel(token_ids, weight):
    B, S = token_ids.shape
    V, D = weight.shape
    P = S // 2
    total = B * S
    n_chunks = (total // NUM_WORKERS) // CHUNK

    idx3 = token_ids.reshape(B, P, 2).transpose(1, 0, 2).astype(jnp.int32)
    idx = idx3.reshape(NUM_WORKERS, n_chunks, CHUNK)

    rows = _gather(idx, weight, total, D, n_chunks)
    pairs = rows.reshape(P, B, 2 * D)
    outc = _transpose(pairs, B, S, D)
    return jnp.transpose(outc, (2, 0, 1))


# speedup vs baseline: 1.0573x; 1.0573x over previous
import functools

import jax
import jax.numpy as jnp
from jax import lax
from jax.experimental import pallas as pl
from jax.experimental.pallas import tpu as pltpu
from jax.experimental.pallas import tpu_sc as plsc

NUM_CORES = 2
NUM_SUBCORES = 16
NUM_WORKERS = NUM_CORES * NUM_SUBCORES

CHUNK = 128
LANES = 16


def _gather(idx, weight, total, D, n_chunks):
    per_worker = total // NUM_WORKERS
    n_pairs = n_chunks // 2
    mesh = plsc.VectorSubcoreMesh(core_axis_name="c", subcore_axis_name="s")

    @functools.partial(
        pl.kernel,
        mesh=mesh,
        out_type=jax.ShapeDtypeStruct((total, D), jnp.float32),
        scratch_types=[
            pltpu.VMEM((n_chunks, CHUNK), jnp.int32),
            pltpu.VMEM((CHUNK, D), jnp.float32),
            pltpu.VMEM((CHUNK, D), jnp.float32),
            pltpu.SemaphoreType.DMA,
            pltpu.SemaphoreType.DMA,
        ],
        compiler_params=pltpu.CompilerParams(use_tc_tiling_on_sc=False),
    )
    def gather_kernel(idx_hbm, table_hbm, out_hbm, idx_v, rows_a, rows_b, sem_a, sem_b):
        wid = lax.axis_index("s") * NUM_CORES + lax.axis_index("c")
        base = wid * per_worker
        pltpu.sync_copy(idx_hbm.at[wid], idx_v)

        pltpu.async_copy(table_hbm.at[idx_v.at[0]], rows_a, sem_a)

        def body(i, carry):
            g = 2 * i
            pltpu.async_copy(table_hbm.at[idx_v.at[g + 1]], rows_b, sem_b)
            pltpu.make_async_copy(table_hbm.at[idx_v.at[g]], rows_a, sem_a).wait()
            pltpu.sync_copy(rows_a, out_hbm.at[pl.ds(base + g * CHUNK, CHUNK)])

            @pl.when(i < n_pairs - 1)
            def _():
                pltpu.async_copy(table_hbm.at[idx_v.at[g + 2]], rows_a, sem_a)

            pltpu.make_async_copy(table_hbm.at[idx_v.at[g + 1]], rows_b, sem_b).wait()
            pltpu.sync_copy(rows_b, out_hbm.at[pl.ds(base + (g + 1) * CHUNK, CHUNK)])
            return carry

        lax.fori_loop(0, n_pairs, body, 0)

    return gather_kernel(idx, weight)


def _transpose(pairs, B, S, D):
    P = S // 2
    nblk = B // 128
    n_units = P * nblk
    units_per_w = n_units // NUM_WORKERS
    mesh = plsc.VectorSubcoreMesh(core_axis_name="c", subcore_axis_name="s")

    @functools.partial(
        pl.kernel,
        mesh=mesh,
        out_type=jax.ShapeDtypeStruct((S, D, B), jnp.float32),
        scratch_types=[
            pltpu.VMEM((128, 128), jnp.float32),
            pltpu.VMEM((128, 128), jnp.float32),
            pltpu.VMEM((D, 128), jnp.float32),
            pltpu.VMEM((D, 128), jnp.float32),
            pltpu.SemaphoreType.DMA,
            pltpu.SemaphoreType.DMA,
            pltpu.SemaphoreType.DMA,
            pltpu.SemaphoreType.DMA,
        ],
        compiler_params=pltpu.CompilerParams(
            use_tc_tiling_on_sc=True, needs_layout_passes=False
        ),
    )
    def ckernel(in_hbm, out_hbm, in_a, in_b, tile_a, tile_b, sia, sib, soa, sob):
        wid = lax.axis_index("s") * NUM_CORES + lax.axis_index("c")
        ubase = wid * units_per_w

        def src(u):
            return in_hbm.at[u // nblk, pl.ds((u % nblk) * 128, 128)]

        pltpu.async_copy(src(ubase), in_a, sia)

        def run_unit(u, stage, sin, other, osin, first):
            p = u // nblk
            blk = u % nblk
            pltpu.make_async_copy(src(u), stage, sin).wait()

            @pl.when(u + 1 < ubase + units_per_w)
            def _():
                pltpu.async_copy(src(u + 1), other, osin)

            rowvs = [
                lax.iota(jnp.int32, 16) + j * LANES for j in range(128 // LANES)
            ]

            for h, (tbuf, osem) in enumerate(((tile_a, soa), (tile_b, sob))):
                if not first:
                    pltpu.make_async_copy(
                        tbuf, out_hbm.at[0, pl.ds(0, D), pl.ds(0, 128)], osem
                    ).wait()

                def trow(c, carry2):
                    colv = jnp.full((16,), h * D + c, dtype=jnp.int32)
                    for j in range(128 // LANES):
                        vals = plsc.load_gather(stage, [rowvs[j], colv])
                        tbuf[c, pl.ds(j * LANES, LANES)] = vals
                    return carry2

                lax.fori_loop(0, D, trow, 0)
                pltpu.async_copy(
                    tbuf,
                    out_hbm.at[2 * p + h, pl.ds(0, D), pl.ds(blk * 128, 128)],
                    osem,
                )

        run_unit(ubase, in_a, sia, in_b, sib, True)

        def body(i, carry):
            u = ubase + 1 + 2 * i
            run_unit(u, in_b, sib, in_a, sia, False)
            run_unit(u + 1, in_a, sia, in_b, sib, False)
            return carry

        lax.fori_loop(0, (units_per_w - 1) // 2, body, 0)
        if units_per_w % 2 == 0:
            run_unit(ubase + units_per_w - 1, in_b, sib, in_a, sia, False)
        for tbuf, osem in ((tile_a, soa), (tile_b, sob)):
            pltpu.make_async_copy(
                tbuf, out_hbm.at[0, pl.ds(0, D), pl.ds(0, 128)], osem
            ).wait()

    return ckernel(pairs)


def kernel(token_ids, weight):
    B, S = token_ids.shape
    V, D = weight.shape
    P = S // 2
    total = B * S
    n_chunks = (total // NUM_WORKERS) // CHUNK

    idx3 = token_ids.reshape(B, P, 2).transpose(1, 0, 2).astype(jnp.int32)
    idx = idx3.reshape(NUM_WORKERS, n_chunks, CHUNK)

    rows = _gather(idx, weight, total, D, n_chunks)
    pairs = rows.reshape(P, B, 2 * D)
    outc = _transpose(pairs, B, S, D)
    return jnp.transpose(outc, (2, 0, 1))


# speedup vs baseline: 1.1945x; 1.1298x over previous
import functools

import jax
import jax.numpy as jnp
from jax import lax
from jax.experimental import pallas as pl
from jax.experimental.pallas import tpu as pltpu
from jax.experimental.pallas import tpu_sc as plsc

NUM_CORES = 2
NUM_SUBCORES = 16
NUM_WORKERS = NUM_CORES * NUM_SUBCORES

CHUNK = 128
LANES = 16


def _gather(idx, weight, total, D, n_chunks):
    per_worker = total // NUM_WORKERS
    n_pairs = n_chunks // 2
    mesh = plsc.VectorSubcoreMesh(core_axis_name="c", subcore_axis_name="s")

    @functools.partial(
        pl.kernel,
        mesh=mesh,
        out_type=jax.ShapeDtypeStruct((total, D), jnp.float32),
        scratch_types=[
            pltpu.VMEM((n_chunks, CHUNK), jnp.int32),
            pltpu.VMEM((CHUNK, D), jnp.float32),
            pltpu.VMEM((CHUNK, D), jnp.float32),
            pltpu.SemaphoreType.DMA,
            pltpu.SemaphoreType.DMA,
        ],
        compiler_params=pltpu.CompilerParams(use_tc_tiling_on_sc=False),
    )
    def gather_kernel(idx_hbm, table_hbm, out_hbm, idx_v, rows_a, rows_b, sem_a, sem_b):
        wid = lax.axis_index("s") * NUM_CORES + lax.axis_index("c")
        base = wid * per_worker
        pltpu.sync_copy(idx_hbm.at[wid], idx_v)

        pltpu.async_copy(table_hbm.at[idx_v.at[0]], rows_a, sem_a)

        def body(i, carry):
            g = 2 * i
            pltpu.async_copy(table_hbm.at[idx_v.at[g + 1]], rows_b, sem_b)
            pltpu.make_async_copy(table_hbm.at[idx_v.at[g]], rows_a, sem_a).wait()
            pltpu.sync_copy(rows_a, out_hbm.at[pl.ds(base + g * CHUNK, CHUNK)])

            @pl.when(i < n_pairs - 1)
            def _():
                pltpu.async_copy(table_hbm.at[idx_v.at[g + 2]], rows_a, sem_a)

            pltpu.make_async_copy(table_hbm.at[idx_v.at[g + 1]], rows_b, sem_b).wait()
            pltpu.sync_copy(rows_b, out_hbm.at[pl.ds(base + (g + 1) * CHUNK, CHUNK)])
            return carry

        lax.fori_loop(0, n_pairs, body, 0)

    return gather_kernel(idx, weight)


def _transpose(pairs, B, S, D):
    P = S // 2
    nblk = B // 128
    n_units = P * nblk
    units_per_w = n_units // NUM_WORKERS
    mesh = plsc.VectorSubcoreMesh(core_axis_name="c", subcore_axis_name="s")

    @functools.partial(
        pl.kernel,
        mesh=mesh,
        out_type=jax.ShapeDtypeStruct((S, D, B), jnp.float32),
        scratch_types=[
            pltpu.VMEM((128, 128), jnp.float32),
            pltpu.VMEM((128, 128), jnp.float32),
            pltpu.VMEM((D, 128), jnp.float32),
            pltpu.VMEM((D, 128), jnp.float32),
            pltpu.SemaphoreType.DMA,
            pltpu.SemaphoreType.DMA,
            pltpu.SemaphoreType.DMA,
            pltpu.SemaphoreType.DMA,
        ],
        compiler_params=pltpu.CompilerParams(
            use_tc_tiling_on_sc=True, needs_layout_passes=False
        ),
    )
    def ckernel(in_hbm, out_hbm, in_a, in_b, tile_a, tile_b, sia, sib, soa, sob):
        wid = lax.axis_index("s") * NUM_CORES + lax.axis_index("c")
        ubase = wid * units_per_w

        def src(u):
            return in_hbm.at[u // nblk, pl.ds((u % nblk) * 128, 128)]

        pltpu.async_copy(src(ubase), in_a, sia)

        def run_unit(u, stage, sin, other, osin, first):
            p = u // nblk
            blk = u % nblk
            pltpu.make_async_copy(src(u), stage, sin).wait()

            @pl.when(u + 1 < ubase + units_per_w)
            def _():
                pltpu.async_copy(src(u + 1), other, osin)

            rowvs = [
                lax.iota(jnp.int32, 16) + j * LANES for j in range(128 // LANES)
            ]

            for h, (tbuf, osem) in enumerate(((tile_a, soa), (tile_b, sob))):
                if not first:
                    pltpu.make_async_copy(
                        tbuf, out_hbm.at[0, pl.ds(0, D), pl.ds(0, 128)], osem
                    ).wait()

                def trow(ci, carry2):
                    c = 2 * ci
                    vals = []
                    for dc in range(2):
                        colv = jnp.full((16,), h * D + c + dc, dtype=jnp.int32)
                        for j in range(128 // LANES):
                            vals.append(plsc.load_gather(stage, [rowvs[j], colv]))
                    for dc in range(2):
                        for j in range(128 // LANES):
                            tbuf[c + dc, pl.ds(j * LANES, LANES)] = vals[
                                dc * (128 // LANES) + j
                            ]
                    return carry2

                lax.fori_loop(0, D // 2, trow, 0)
                pltpu.async_copy(
                    tbuf,
                    out_hbm.at[2 * p + h, pl.ds(0, D), pl.ds(blk * 128, 128)],
                    osem,
                )

        run_unit(ubase, in_a, sia, in_b, sib, True)

        def body(i, carry):
            u = ubase + 1 + 2 * i
            run_unit(u, in_b, sib, in_a, sia, False)
            run_unit(u + 1, in_a, sia, in_b, sib, False)
            return carry

        lax.fori_loop(0, (units_per_w - 1) // 2, body, 0)
        if units_per_w % 2 == 0:
            run_unit(ubase + units_per_w - 1, in_b, sib, in_a, sia, False)
        for tbuf, osem in ((tile_a, soa), (tile_b, sob)):
            pltpu.make_async_copy(
                tbuf, out_hbm.at[0, pl.ds(0, D), pl.ds(0, 128)], osem
            ).wait()

    return ckernel(pairs)


def kernel(token_ids, weight):
    B, S = token_ids.shape
    V, D = weight.shape
    P = S // 2
    total = B * S
    n_chunks = (total // NUM_WORKERS) // CHUNK

    idx3 = token_ids.reshape(B, P, 2).transpose(1, 0, 2).astype(jnp.int32)
    idx = idx3.reshape(NUM_WORKERS, n_chunks, CHUNK)

    rows = _gather(idx, weight, total, D, n_chunks)
    pairs = rows.reshape(P, B, 2 * D)
    outc = _transpose(pairs, B, S, D)
    return jnp.transpose(outc, (2, 0, 1))


# speedup vs baseline: 1.7834x; 1.4931x over previous
import functools

import jax
import jax.numpy as jnp
from jax import lax
from jax.experimental import pallas as pl
from jax.experimental.pallas import tpu as pltpu
from jax.experimental.pallas import tpu_sc as plsc

NUM_CORES = 2
NUM_SUBCORES = 16
NUM_WORKERS = NUM_CORES * NUM_SUBCORES

CHUNK = 128


def kernel(token_ids, weight):
    B, S = token_ids.shape
    V, D = weight.shape
    total = B * S
    per_worker = total // NUM_WORKERS
    n_chunks = per_worker // CHUNK
    n_pairs = n_chunks // 2

    idx = token_ids.reshape(NUM_WORKERS, n_chunks, CHUNK).astype(jnp.int32)

    mesh = plsc.VectorSubcoreMesh(core_axis_name="c", subcore_axis_name="s")

    @functools.partial(
        pl.kernel,
        mesh=mesh,
        out_type=jax.ShapeDtypeStruct((total, D), jnp.float32),
        scratch_types=[
            pltpu.VMEM((n_chunks, CHUNK), jnp.int32),
            pltpu.VMEM((CHUNK, D), jnp.float32),
            pltpu.VMEM((CHUNK, D), jnp.float32),
            pltpu.SemaphoreType.DMA,
            pltpu.SemaphoreType.DMA,
        ],
        compiler_params=pltpu.CompilerParams(use_tc_tiling_on_sc=False),
    )
    def gather_kernel(idx_hbm, table_hbm, out_hbm, idx_v, rows_a, rows_b, sem_a, sem_b):
        wid = lax.axis_index("s") * NUM_CORES + lax.axis_index("c")
        base = wid * per_worker
        pltpu.sync_copy(idx_hbm.at[wid], idx_v)

        pltpu.async_copy(table_hbm.at[idx_v.at[0]], rows_a, sem_a)

        def body(i, carry):
            g = 2 * i
            pltpu.async_copy(table_hbm.at[idx_v.at[g + 1]], rows_b, sem_b)
            pltpu.make_async_copy(table_hbm.at[idx_v.at[g]], rows_a, sem_a).wait()
            pltpu.sync_copy(rows_a, out_hbm.at[pl.ds(base + g * CHUNK, CHUNK)])

            @pl.when(i < n_pairs - 1)
            def _():
                pltpu.async_copy(table_hbm.at[idx_v.at[g + 2]], rows_a, sem_a)

            pltpu.make_async_copy(table_hbm.at[idx_v.at[g + 1]], rows_b, sem_b).wait()
            pltpu.sync_copy(rows_b, out_hbm.at[pl.ds(base + (g + 1) * CHUNK, CHUNK)])
            return carry

        lax.fori_loop(0, n_pairs, body, 0)

    out = gather_kernel(idx, weight)
    return out.reshape(B, S, D)
